# baseline (speedup 1.0000x reference)
import functools

import jax
import jax.numpy as jnp
import numpy as np
from jax import lax
from jax.experimental import pallas as pl
from jax.experimental.pallas import tpu as pltpu
from jax.experimental.pallas import tpu_sc as plsc

B = 16384
NUM_NUMERICAL = 13
N_CAT = 26
VOCAB = 100000
D_TOKEN = 32
N_TOK = 1 + NUM_NUMERICAL + N_CAT
N_NC = 1 + NUM_NUMERICAL

NUM_PAD = 16
CH = 16
LANES = 16

RV = 1024
RNB = (VOCAB + RV - 1) // RV
FSTRIDE = RNB * RV // 4
PACKED_ROWS = N_CAT * FSTRIDE


def _repack_body(t_ref, o_ref):
    x = t_ref[0]
    for t in range(RV // 128):
        xt = x[:, 128 * t:128 * (t + 1)]
        for q in range(4):
            y = jnp.transpose(xt[:, 32 * q:32 * (q + 1)], (1, 0))
            z = y.reshape(8, 4, 32)
            oq = jnp.concatenate(
                [z[:, 0, :], z[:, 1, :], z[:, 2, :], z[:, 3, :]], axis=1)
            o_ref[32 * t + 8 * q:32 * t + 8 * q + 8, :] = oq


def _repack_table(tbl_t):
    return pl.pallas_call(
        _repack_body,
        grid=(N_CAT, RNB),
        in_specs=[pl.BlockSpec((1, D_TOKEN, RV), lambda c, v: (c, 0, v))],
        out_specs=pl.BlockSpec((RV // 4, 128), lambda c, v: (c * RNB + v, 0)),
        out_shape=jax.ShapeDtypeStruct((PACKED_ROWS, 128), jnp.float32),
    )(tbl_t)


def _tokenizer_body(x_num_hbm, x_cat_hbm, cls_hbm, w_hbm, bias_hbm,
                    tables_hbm, crow_hbm, cpat_hbm, npat_hbm, out_hbm,
                    xcat_v, row_v, pq_v, crow_v, cpat_v, npat_v,
                    cdidx_v, ndidx_v, xnum_v, w_v, b_v, cls_v,
                    gath_v, cat_v, numcls_v, gsem, ssem):
    info = plsc.get_sparse_core_info()
    nc, ns = info.num_cores, info.num_subcores
    nw = nc * ns
    rows_per_w = B // nw
    nch = rows_per_w // CH
    ids_per_ch = CH * N_CAT
    nc_per_ch = CH * N_NC

    wid = lax.axis_index("s") * nc + lax.axis_index("c")

    pltpu.sync_copy(w_hbm, w_v)
    pltpu.sync_copy(bias_hbm, b_v)
    pltpu.sync_copy(cls_hbm, cls_v)
    pltpu.sync_copy(crow_hbm, crow_v)
    pltpu.sync_copy(cpat_hbm, cpat_v)
    pltpu.sync_copy(npat_hbm, npat_v)

    def chunk_body(k, carry):
        base = (wid * nch + k) * CH

        pltpu.sync_copy(x_cat_hbm.at[pl.ds(base * N_CAT, ids_per_ch)], xcat_v)
        pltpu.sync_copy(x_num_hbm.at[pl.ds(base * NUM_PAD, CH * NUM_PAD)],
                        xnum_v)

        obase = base * N_TOK

        def idx_body(i, _):
            s = pl.ds(i * LANES, LANES)
            v = xcat_v[s]
            row_v[s] = (crow_v[s]
                        + ((v >> 7) << 5)
                        + (((v >> 5) & 3) << 3)
                        + ((v & 31) >> 2))
            pq_v[s] = (v & 3) << 5
            cdidx_v[s] = cpat_v[s] + obase
            return 0

        def nd_body(i, _):
            s = pl.ds(i * LANES, LANES)
            ndidx_v[s] = npat_v[s] + obase
            return 0

        lax.fori_loop(0, ids_per_ch // LANES, idx_body, 0)
        lax.fori_loop(0, nc_per_ch // LANES, nd_body, 0)

        pltpu.async_copy(tables_hbm.at[row_v], gath_v, gsem)

        cls0 = cls_v[pl.ds(0, LANES)]
        cls1 = cls_v[pl.ds(LANES, LANES)]

        def num_body(bi, _):
            r = bi * N_NC
            numcls_v[r, pl.ds(0, LANES)] = cls0
            numcls_v[r, pl.ds(LANES, LANES)] = cls1
            xrow = xnum_v[pl.ds(bi * NUM_PAD, NUM_PAD)]
            for j in range(NUM_NUMERICAL):
                xs = xrow[j]
                numcls_v[r + 1 + j, pl.ds(0, LANES)] = (
                    xs * w_v[j, pl.ds(0, LANES)] + b_v[j, pl.ds(0, LANES)])
                numcls_v[r + 1 + j, pl.ds(LANES, LANES)] = (
                    xs * w_v[j, pl.ds(LANES, LANES)] + b_v[j, pl.ds(LANES, LANES)])
            return 0

        lax.fori_loop(0, CH, num_body, 0)

        pltpu.async_copy(numcls_v, out_hbm.at[ndidx_v], ssem)

        pltpu.make_async_copy(tables_hbm.at[row_v], gath_v, gsem).wait()

        def ext_body(g, _):
            offs = pq_v[pl.ds(g * LANES, LANES)]
            for j in range(LANES):
                tok = g * LANES + j
                o = offs[j]
                cat_v[tok, pl.ds(0, LANES)] = gath_v[tok, pl.ds(o, LANES)]
                cat_v[tok, pl.ds(LANES, LANES)] = (
                    gath_v[tok, pl.ds(o + LANES, LANES)])
            return 0

        lax.fori_loop(0, ids_per_ch // LANES, ext_body, 0)

        pltpu.async_copy(cat_v, out_hbm.at[cdidx_v], ssem)

        pltpu.make_async_copy(numcls_v, out_hbm.at[ndidx_v], ssem).wait()
        pltpu.make_async_copy(cat_v, out_hbm.at[cdidx_v], ssem).wait()
        return 0

    lax.fori_loop(0, nch, chunk_body, 0)


@functools.partial(jax.jit, static_argnames=())
def kernel(x_num, x_cat, cls_token, num_weights, num_biases, cat_tables):
    tbl_t = jnp.transpose(cat_tables, (0, 2, 1))
    tables_packed = _repack_table(tbl_t)

    x_nump = jnp.concatenate(
        [x_num, jnp.zeros((B, NUM_PAD - NUM_NUMERICAL), jnp.float32)], axis=1
    ).reshape(B * NUM_PAD)
    x_cat_flat = x_cat.astype(jnp.int32).reshape(B * N_CAT)
    cls_flat = cls_token.reshape(D_TOKEN)

    p = np.arange(CH * N_CAT, dtype=np.int32)
    crow_np = (p % N_CAT) * FSTRIDE
    cpat_np = (p // N_CAT) * N_TOK + N_NC + (p % N_CAT)
    q = np.arange(CH * N_NC, dtype=np.int32)
    npat_np = (q // N_NC) * N_TOK + (q % N_NC)

    mesh = plsc.VectorSubcoreMesh(core_axis_name="c", subcore_axis_name="s")
    run = pl.kernel(
        _tokenizer_body,
        out_type=jax.ShapeDtypeStruct((B * N_TOK, D_TOKEN), jnp.float32),
        mesh=mesh,
        compiler_params=pltpu.CompilerParams(use_tc_tiling_on_sc=False),
        scratch_types=[
            pltpu.VMEM((CH * N_CAT,), jnp.int32),
            pltpu.VMEM((CH * N_CAT,), jnp.int32),
            pltpu.VMEM((CH * N_CAT,), jnp.int32),
            pltpu.VMEM((CH * N_CAT,), jnp.int32),
            pltpu.VMEM((CH * N_CAT,), jnp.int32),
            pltpu.VMEM((CH * N_NC,), jnp.int32),
            pltpu.VMEM((CH * N_CAT,), jnp.int32),
            pltpu.VMEM((CH * N_NC,), jnp.int32),
            pltpu.VMEM((CH * NUM_PAD,), jnp.float32),
            pltpu.VMEM((NUM_NUMERICAL, D_TOKEN), jnp.float32),
            pltpu.VMEM((NUM_NUMERICAL, D_TOKEN), jnp.float32),
            pltpu.VMEM((D_TOKEN,), jnp.float32),
            pltpu.VMEM((CH * N_CAT, 128), jnp.float32),
            pltpu.VMEM((CH * N_CAT, D_TOKEN), jnp.float32),
            pltpu.VMEM((CH * N_NC, D_TOKEN), jnp.float32),
            pltpu.SemaphoreType.DMA,
            pltpu.SemaphoreType.DMA,
        ],
    )
    out_flat = run(x_nump, x_cat_flat, cls_flat, num_weights, num_biases,
                   tables_packed, jnp.asarray(crow_np), jnp.asarray(cpat_np),
                   jnp.asarray(npat_np))
    return out_flat.reshape(B, N_TOK, D_TOKEN)


# speedup vs baseline: 4.5740x; 4.5740x over previous
import functools

import jax
import jax.numpy as jnp
import numpy as np
from jax import lax
from jax.experimental import pallas as pl
from jax.experimental.pallas import tpu as pltpu
from jax.experimental.pallas import tpu_sc as plsc

B = 16384
NUM_NUMERICAL = 13
N_CAT = 26
VOCAB = 100000
D_TOKEN = 32
N_TOK = 1 + NUM_NUMERICAL + N_CAT
N_NC = 1 + NUM_NUMERICAL

NUM_PAD = 16
CH = 16
LANES = 16

RV = 512
FSTRIDE = 25088
RNB = FSTRIDE // RV
PACKED_ROWS = N_CAT * FSTRIDE


def _repack_body(t0, t1, t2, t3, o_ref):
    ys = [jnp.transpose(t[0], (1, 0)) for t in (t0, t1, t2, t3)]
    o_ref[...] = jnp.concatenate(ys, axis=1)


def _repack_table(tbl_t):
    specs = [
        pl.BlockSpec((1, D_TOKEN, RV),
                     lambda c, v, p=p: (c, 0, p * RNB + v))
        for p in range(4)
    ]
    return pl.pallas_call(
        _repack_body,
        grid=(N_CAT, RNB),
        in_specs=specs,
        out_specs=pl.BlockSpec((RV, 128), lambda c, v: (c * RNB + v, 0)),
        out_shape=jax.ShapeDtypeStruct((PACKED_ROWS, 128), jnp.float32),
    )(tbl_t, tbl_t, tbl_t, tbl_t)


def _tokenizer_body(x_num_hbm, x_cat_hbm, cls_hbm, w_hbm, bias_hbm,
                    tables_hbm, crow_hbm, cpat_hbm, npat_hbm, out_hbm,
                    xcat_v, row_v, pq_v, crow_v, cpat_v, npat_v,
                    cdidx_v, ndidx_v, xnum_v, w_v, b_v, cls_v,
                    gath_v, cat_v, numcls_v, gsem, ssem):
    info = plsc.get_sparse_core_info()
    nc, ns = info.num_cores, info.num_subcores
    nw = nc * ns
    rows_per_w = B // nw
    nch = rows_per_w // CH
    ids_per_ch = CH * N_CAT
    nc_per_ch = CH * N_NC

    wid = lax.axis_index("s") * nc + lax.axis_index("c")

    pltpu.sync_copy(w_hbm, w_v)
    pltpu.sync_copy(bias_hbm, b_v)
    pltpu.sync_copy(cls_hbm, cls_v)
    pltpu.sync_copy(crow_hbm, crow_v)
    pltpu.sync_copy(cpat_hbm, cpat_v)
    pltpu.sync_copy(npat_hbm, npat_v)

    def chunk_body(k, carry):
        base = (wid * nch + k) * CH

        pltpu.sync_copy(x_cat_hbm.at[pl.ds(base * N_CAT, ids_per_ch)], xcat_v)
        pltpu.sync_copy(x_num_hbm.at[pl.ds(base * NUM_PAD, CH * NUM_PAD)],
                        xnum_v)

        obase = base * N_TOK

        def idx_body(i, _):
            s = pl.ds(i * LANES, LANES)
            v = xcat_v[s]
            one = jnp.full((LANES,), 1, jnp.int32)
            zero = jnp.full((LANES,), 0, jnp.int32)
            p = (jnp.where(v >= FSTRIDE, one, zero)
                 + jnp.where(v >= 2 * FSTRIDE, one, zero)
                 + jnp.where(v >= 3 * FSTRIDE, one, zero))
            row_v[s] = crow_v[s] + v - p * FSTRIDE
            pq_v[s] = p << 5
            cdidx_v[s] = cpat_v[s] + obase
            return 0

        def nd_body(i, _):
            s = pl.ds(i * LANES, LANES)
            ndidx_v[s] = npat_v[s] + obase
            return 0

        lax.fori_loop(0, ids_per_ch // LANES, idx_body, 0)
        lax.fori_loop(0, nc_per_ch // LANES, nd_body, 0)

        pltpu.async_copy(tables_hbm.at[row_v], gath_v, gsem)

        cls0 = cls_v[pl.ds(0, LANES)]
        cls1 = cls_v[pl.ds(LANES, LANES)]

        def num_body(bi, _):
            r = bi * N_NC
            numcls_v[r, pl.ds(0, LANES)] = cls0
            numcls_v[r, pl.ds(LANES, LANES)] = cls1
            xrow = xnum_v[pl.ds(bi * NUM_PAD, NUM_PAD)]
            for j in range(NUM_NUMERICAL):
                xs = xrow[j]
                numcls_v[r + 1 + j, pl.ds(0, LANES)] = (
                    xs * w_v[j, pl.ds(0, LANES)] + b_v[j, pl.ds(0, LANES)])
                numcls_v[r + 1 + j, pl.ds(LANES, LANES)] = (
                    xs * w_v[j, pl.ds(LANES, LANES)] + b_v[j, pl.ds(LANES, LANES)])
            return 0

        lax.fori_loop(0, CH, num_body, 0)

        pltpu.async_copy(numcls_v, out_hbm.at[ndidx_v], ssem)

        pltpu.make_async_copy(tables_hbm.at[row_v], gath_v, gsem).wait()

        def ext_body(g, _):
            offs = pq_v[pl.ds(g * LANES, LANES)]
            for j in range(LANES):
                tok = g * LANES + j
                o = offs[j]
                cat_v[tok, pl.ds(0, LANES)] = gath_v[tok, pl.ds(o, LANES)]
                cat_v[tok, pl.ds(LANES, LANES)] = (
                    gath_v[tok, pl.ds(o + LANES, LANES)])
            return 0

        lax.fori_loop(0, ids_per_ch // LANES, ext_body, 0)

        pltpu.async_copy(cat_v, out_hbm.at[cdidx_v], ssem)

        pltpu.make_async_copy(numcls_v, out_hbm.at[ndidx_v], ssem).wait()
        pltpu.make_async_copy(cat_v, out_hbm.at[cdidx_v], ssem).wait()
        return 0

    lax.fori_loop(0, nch, chunk_body, 0)


@functools.partial(jax.jit, static_argnames=())
def kernel(x_num, x_cat, cls_token, num_weights, num_biases, cat_tables):
    tbl_t = jnp.transpose(cat_tables, (0, 2, 1))
    tables_packed = _repack_table(tbl_t)

    x_nump = jnp.concatenate(
        [x_num, jnp.zeros((B, NUM_PAD - NUM_NUMERICAL), jnp.float32)], axis=1
    ).reshape(B * NUM_PAD)
    x_cat_flat = x_cat.astype(jnp.int32).reshape(B * N_CAT)
    cls_flat = cls_token.reshape(D_TOKEN)

    p = np.arange(CH * N_CAT, dtype=np.int32)
    crow_np = (p % N_CAT) * FSTRIDE
    cpat_np = (p // N_CAT) * N_TOK + N_NC + (p % N_CAT)
    q = np.arange(CH * N_NC, dtype=np.int32)
    npat_np = (q // N_NC) * N_TOK + (q % N_NC)

    mesh = plsc.VectorSubcoreMesh(core_axis_name="c", subcore_axis_name="s")
    run = pl.kernel(
        _tokenizer_body,
        out_type=jax.ShapeDtypeStruct((B * N_TOK, D_TOKEN), jnp.float32),
        mesh=mesh,
        compiler_params=pltpu.CompilerParams(use_tc_tiling_on_sc=False),
        scratch_types=[
            pltpu.VMEM((CH * N_CAT,), jnp.int32),
            pltpu.VMEM((CH * N_CAT,), jnp.int32),
            pltpu.VMEM((CH * N_CAT,), jnp.int32),
            pltpu.VMEM((CH * N_CAT,), jnp.int32),
            pltpu.VMEM((CH * N_CAT,), jnp.int32),
            pltpu.VMEM((CH * N_NC,), jnp.int32),
            pltpu.VMEM((CH * N_CAT,), jnp.int32),
            pltpu.VMEM((CH * N_NC,), jnp.int32),
            pltpu.VMEM((CH * NUM_PAD,), jnp.float32),
            pltpu.VMEM((NUM_NUMERICAL, D_TOKEN), jnp.float32),
            pltpu.VMEM((NUM_NUMERICAL, D_TOKEN), jnp.float32),
            pltpu.VMEM((D_TOKEN,), jnp.float32),
            pltpu.VMEM((CH * N_CAT, 128), jnp.float32),
            pltpu.VMEM((CH * N_CAT, D_TOKEN), jnp.float32),
            pltpu.VMEM((CH * N_NC, D_TOKEN), jnp.float32),
            pltpu.SemaphoreType.DMA,
            pltpu.SemaphoreType.DMA,
        ],
    )
    out_flat = run(x_nump, x_cat_flat, cls_flat, num_weights, num_biases,
                   tables_packed, jnp.asarray(crow_np), jnp.asarray(cpat_np),
                   jnp.asarray(npat_np))
    return out_flat.reshape(B, N_TOK, D_TOKEN)


# speedup vs baseline: 6.1350x; 1.3413x over previous
import functools

import jax
import jax.numpy as jnp
import numpy as np
from jax import lax
from jax.experimental import pallas as pl
from jax.experimental.pallas import tpu as pltpu
from jax.experimental.pallas import tpu_sc as plsc

B = 16384
NUM_NUMERICAL = 13
N_CAT = 26
VOCAB = 100000
D_TOKEN = 32
N_TOK = 1 + NUM_NUMERICAL + N_CAT
N_NC = 1 + NUM_NUMERICAL

NUM_PAD = 16
CH = 16
LANES = 16

RV = 3584
FSTRIDE = 25088
RNB = FSTRIDE // RV
PACKED_ROWS = N_CAT * FSTRIDE


def _repack_body(t0, t1, t2, t3, o_ref):
    ys = [jnp.transpose(t[0], (1, 0)) for t in (t0, t1, t2, t3)]
    o_ref[...] = jnp.concatenate(ys, axis=1)


def _repack_table(tbl_t):
    specs = [
        pl.BlockSpec((1, D_TOKEN, RV),
                     lambda c, v, p=p: (c, 0, p * RNB + v))
        for p in range(4)
    ]
    return pl.pallas_call(
        _repack_body,
        grid=(N_CAT, RNB),
        in_specs=specs,
        out_specs=pl.BlockSpec((RV, 128), lambda c, v: (c * RNB + v, 0)),
        out_shape=jax.ShapeDtypeStruct((PACKED_ROWS, 128), jnp.float32),
    )(tbl_t, tbl_t, tbl_t, tbl_t)


def _tokenizer_body(x_num_hbm, x_cat_hbm, cls_hbm, w_hbm, bias_hbm,
                    tables_hbm, crow_hbm, cpat_hbm, npat_hbm, out_hbm,
                    xcat_v, row_v, pq_v, crow_v, cpat_v, npat_v,
                    cdidx_v, ndidx_v, xnum_v, w_v, b_v, cls_v,
                    gath_v, cat_v, numcls_v, gsem, ssem):
    info = plsc.get_sparse_core_info()
    nc, ns = info.num_cores, info.num_subcores
    nw = nc * ns
    rows_per_w = B // nw
    nch = rows_per_w // CH
    ids_per_ch = CH * N_CAT
    nc_per_ch = CH * N_NC

    wid = lax.axis_index("s") * nc + lax.axis_index("c")

    pltpu.sync_copy(w_hbm, w_v)
    pltpu.sync_copy(bias_hbm, b_v)
    pltpu.sync_copy(cls_hbm, cls_v)
    pltpu.sync_copy(crow_hbm, crow_v)
    pltpu.sync_copy(cpat_hbm, cpat_v)
    pltpu.sync_copy(npat_hbm, npat_v)

    def chunk_body(k, carry):
        base = (wid * nch + k) * CH

        pltpu.sync_copy(x_cat_hbm.at[pl.ds(base * N_CAT, ids_per_ch)], xcat_v)
        pltpu.sync_copy(x_num_hbm.at[pl.ds(base * NUM_PAD, CH * NUM_PAD)],
                        xnum_v)

        obase = base * N_TOK

        def idx_body(i, _):
            s = pl.ds(i * LANES, LANES)
            v = xcat_v[s]
            one = jnp.full((LANES,), 1, jnp.int32)
            zero = jnp.full((LANES,), 0, jnp.int32)
            p = (jnp.where(v >= FSTRIDE, one, zero)
                 + jnp.where(v >= 2 * FSTRIDE, one, zero)
                 + jnp.where(v >= 3 * FSTRIDE, one, zero))
            row_v[s] = crow_v[s] + v - p * FSTRIDE
            pq_v[s] = p << 5
            cdidx_v[s] = cpat_v[s] + obase
            return 0

        def nd_body(i, _):
            s = pl.ds(i * LANES, LANES)
            ndidx_v[s] = npat_v[s] + obase
            return 0

        lax.fori_loop(0, ids_per_ch // LANES, idx_body, 0)
        lax.fori_loop(0, nc_per_ch // LANES, nd_body, 0)

        pltpu.async_copy(tables_hbm.at[row_v], gath_v, gsem)

        cls0 = cls_v[pl.ds(0, LANES)]
        cls1 = cls_v[pl.ds(LANES, LANES)]

        def num_body(bi, _):
            r = bi * N_NC
            numcls_v[r, pl.ds(0, LANES)] = cls0
            numcls_v[r, pl.ds(LANES, LANES)] = cls1
            xrow = xnum_v[pl.ds(bi * NUM_PAD, NUM_PAD)]
            for j in range(NUM_NUMERICAL):
                xs = xrow[j]
                numcls_v[r + 1 + j, pl.ds(0, LANES)] = (
                    xs * w_v[j, pl.ds(0, LANES)] + b_v[j, pl.ds(0, LANES)])
                numcls_v[r + 1 + j, pl.ds(LANES, LANES)] = (
                    xs * w_v[j, pl.ds(LANES, LANES)] + b_v[j, pl.ds(LANES, LANES)])
            return 0

        lax.fori_loop(0, CH, num_body, 0)

        pltpu.async_copy(numcls_v, out_hbm.at[ndidx_v], ssem)

        pltpu.make_async_copy(tables_hbm.at[row_v], gath_v, gsem).wait()

        def ext_body(g, _):
            offs = pq_v[pl.ds(g * LANES, LANES)]
            for j in range(LANES):
                tok = g * LANES + j
                o = offs[j]
                cat_v[tok, pl.ds(0, LANES)] = gath_v[tok, pl.ds(o, LANES)]
                cat_v[tok, pl.ds(LANES, LANES)] = (
                    gath_v[tok, pl.ds(o + LANES, LANES)])
            return 0

        lax.fori_loop(0, ids_per_ch // LANES, ext_body, 0)

        pltpu.async_copy(cat_v, out_hbm.at[cdidx_v], ssem)

        pltpu.make_async_copy(numcls_v, out_hbm.at[ndidx_v], ssem).wait()
        pltpu.make_async_copy(cat_v, out_hbm.at[cdidx_v], ssem).wait()
        return 0

    lax.fori_loop(0, nch, chunk_body, 0)


@functools.partial(jax.jit, static_argnames=())
def kernel(x_num, x_cat, cls_token, num_weights, num_biases, cat_tables):
    tbl_t = jnp.transpose(cat_tables, (0, 2, 1))
    tables_packed = _repack_table(tbl_t)

    x_nump = jnp.concatenate(
        [x_num, jnp.zeros((B, NUM_PAD - NUM_NUMERICAL), jnp.float32)], axis=1
    ).reshape(B * NUM_PAD)
    x_cat_flat = x_cat.astype(jnp.int32).reshape(B * N_CAT)
    cls_flat = cls_token.reshape(D_TOKEN)

    p = np.arange(CH * N_CAT, dtype=np.int32)
    crow_np = (p % N_CAT) * FSTRIDE
    cpat_np = (p // N_CAT) * N_TOK + N_NC + (p % N_CAT)
    q = np.arange(CH * N_NC, dtype=np.int32)
    npat_np = (q // N_NC) * N_TOK + (q % N_NC)

    mesh = plsc.VectorSubcoreMesh(core_axis_name="c", subcore_axis_name="s")
    run = pl.kernel(
        _tokenizer_body,
        out_type=jax.ShapeDtypeStruct((B * N_TOK, D_TOKEN), jnp.float32),
        mesh=mesh,
        compiler_params=pltpu.CompilerParams(use_tc_tiling_on_sc=False),
        scratch_types=[
            pltpu.VMEM((CH * N_CAT,), jnp.int32),
            pltpu.VMEM((CH * N_CAT,), jnp.int32),
            pltpu.VMEM((CH * N_CAT,), jnp.int32),
            pltpu.VMEM((CH * N_CAT,), jnp.int32),
            pltpu.VMEM((CH * N_CAT,), jnp.int32),
            pltpu.VMEM((CH * N_NC,), jnp.int32),
            pltpu.VMEM((CH * N_CAT,), jnp.int32),
            pltpu.VMEM((CH * N_NC,), jnp.int32),
            pltpu.VMEM((CH * NUM_PAD,), jnp.float32),
            pltpu.VMEM((NUM_NUMERICAL, D_TOKEN), jnp.float32),
            pltpu.VMEM((NUM_NUMERICAL, D_TOKEN), jnp.float32),
            pltpu.VMEM((D_TOKEN,), jnp.float32),
            pltpu.VMEM((CH * N_CAT, 128), jnp.float32),
            pltpu.VMEM((CH * N_CAT, D_TOKEN), jnp.float32),
            pltpu.VMEM((CH * N_NC, D_TOKEN), jnp.float32),
            pltpu.SemaphoreType.DMA,
            pltpu.SemaphoreType.DMA,
        ],
    )
    out_flat = run(x_nump, x_cat_flat, cls_flat, num_weights, num_biases,
                   tables_packed, jnp.asarray(crow_np), jnp.asarray(cpat_np),
                   jnp.asarray(npat_np))
    return out_flat.reshape(B, N_TOK, D_TOKEN)


# speedup vs baseline: 6.8698x; 1.1198x over previous
import functools

import jax
import jax.numpy as jnp
import numpy as np
from jax import lax
from jax.experimental import pallas as pl
from jax.experimental.pallas import tpu as pltpu
from jax.experimental.pallas import tpu_sc as plsc

B = 16384
NUM_NUMERICAL = 13
N_CAT = 26
VOCAB = 100000
D_TOKEN = 32
N_TOK = 1 + NUM_NUMERICAL + N_CAT
N_NC = 1 + NUM_NUMERICAL

NUM_PAD = 16
CH = 64
LANES = 16

RV = 3584
FSTRIDE = 25088
RNB = FSTRIDE // RV
PACKED_ROWS = N_CAT * FSTRIDE * 4


def _repack_body(t0, t1, t2, t3, o_ref):
    ys = [jnp.transpose(t[0], (1, 0)) for t in (t0, t1, t2, t3)]
    o_ref[...] = jnp.concatenate(ys, axis=1)


def _repack_table(tbl_t):
    specs = [
        pl.BlockSpec((1, D_TOKEN, RV),
                     lambda c, v, p=p: (c, 0, p * RNB + v))
        for p in range(4)
    ]
    return pl.pallas_call(
        _repack_body,
        grid=(N_CAT, RNB),
        in_specs=specs,
        out_specs=pl.BlockSpec((RV, 128), lambda c, v: (c * RNB + v, 0)),
        out_shape=jax.ShapeDtypeStruct((PACKED_ROWS // 4, 128), jnp.float32),
    )(tbl_t, tbl_t, tbl_t, tbl_t)


def _tokenizer_body(x_num_hbm, x_cat_hbm, cls_hbm, w_hbm, bias_hbm,
                    tables_hbm, crow_hbm, cpat_hbm, npat_hbm, out_hbm,
                    xcat_v, row_v, crow_v, cpat_v, npat_v,
                    cdidx_v, ndidx_v, xnum_v, w_v, b_v, cls_v,
                    cat_v, numcls_v, gsem, ssem):
    info = plsc.get_sparse_core_info()
    nc, ns = info.num_cores, info.num_subcores
    nw = nc * ns
    rows_per_w = B // nw
    nch = rows_per_w // CH
    ids_per_ch = CH * N_CAT
    nc_per_ch = CH * N_NC

    wid = lax.axis_index("s") * nc + lax.axis_index("c")

    pltpu.sync_copy(w_hbm, w_v)
    pltpu.sync_copy(bias_hbm, b_v)
    pltpu.sync_copy(cls_hbm, cls_v)
    pltpu.sync_copy(crow_hbm, crow_v)
    pltpu.sync_copy(cpat_hbm, cpat_v)
    pltpu.sync_copy(npat_hbm, npat_v)

    def chunk_body(k, carry):
        base = (wid * nch + k) * CH

        pltpu.sync_copy(x_cat_hbm.at[pl.ds(base * N_CAT, ids_per_ch)], xcat_v)
        pltpu.sync_copy(x_num_hbm.at[pl.ds(base * NUM_PAD, CH * NUM_PAD)],
                        xnum_v)

        obase = base * N_TOK

        def idx_body(i, _):
            s = pl.ds(i * LANES, LANES)
            v = xcat_v[s]
            one = jnp.full((LANES,), 1, jnp.int32)
            zero = jnp.full((LANES,), 0, jnp.int32)
            p = (jnp.where(v >= FSTRIDE, one, zero)
                 + jnp.where(v >= 2 * FSTRIDE, one, zero)
                 + jnp.where(v >= 3 * FSTRIDE, one, zero))
            row_v[s] = crow_v[s] + ((v - p * FSTRIDE) << 2) + p
            cdidx_v[s] = cpat_v[s] + obase
            return 0

        def nd_body(i, _):
            s = pl.ds(i * LANES, LANES)
            ndidx_v[s] = npat_v[s] + obase
            return 0

        lax.fori_loop(0, ids_per_ch // LANES, idx_body, 0)
        lax.fori_loop(0, nc_per_ch // LANES, nd_body, 0)

        pltpu.async_copy(tables_hbm.at[row_v], cat_v, gsem)

        cls0 = cls_v[pl.ds(0, LANES)]
        cls1 = cls_v[pl.ds(LANES, LANES)]

        def num_body(bi, _):
            r = bi * N_NC
            numcls_v[r, pl.ds(0, LANES)] = cls0
            numcls_v[r, pl.ds(LANES, LANES)] = cls1
            xrow = xnum_v[pl.ds(bi * NUM_PAD, NUM_PAD)]
            for j in range(NUM_NUMERICAL):
                xs = xrow[j]
                numcls_v[r + 1 + j, pl.ds(0, LANES)] = (
                    xs * w_v[j, pl.ds(0, LANES)] + b_v[j, pl.ds(0, LANES)])
                numcls_v[r + 1 + j, pl.ds(LANES, LANES)] = (
                    xs * w_v[j, pl.ds(LANES, LANES)] + b_v[j, pl.ds(LANES, LANES)])
            return 0

        lax.fori_loop(0, CH, num_body, 0)

        pltpu.async_copy(numcls_v, out_hbm.at[ndidx_v], ssem)

        pltpu.make_async_copy(tables_hbm.at[row_v], cat_v, gsem).wait()
        pltpu.async_copy(cat_v, out_hbm.at[cdidx_v], ssem)

        pltpu.make_async_copy(numcls_v, out_hbm.at[ndidx_v], ssem).wait()
        pltpu.make_async_copy(cat_v, out_hbm.at[cdidx_v], ssem).wait()
        return 0

    lax.fori_loop(0, nch, chunk_body, 0)


@functools.partial(jax.jit, static_argnames=())
def kernel(x_num, x_cat, cls_token, num_weights, num_biases, cat_tables):
    tbl_t = jnp.transpose(cat_tables, (0, 2, 1))
    tables_packed = _repack_table(tbl_t).reshape(PACKED_ROWS, D_TOKEN)

    x_nump = jnp.concatenate(
        [x_num, jnp.zeros((B, NUM_PAD - NUM_NUMERICAL), jnp.float32)], axis=1
    ).reshape(B * NUM_PAD)
    x_cat_flat = x_cat.astype(jnp.int32).reshape(B * N_CAT)
    cls_flat = cls_token.reshape(D_TOKEN)

    p = np.arange(CH * N_CAT, dtype=np.int32)
    crow_np = (p % N_CAT) * (4 * FSTRIDE)
    cpat_np = (p // N_CAT) * N_TOK + N_NC + (p % N_CAT)
    q = np.arange(CH * N_NC, dtype=np.int32)
    npat_np = (q // N_NC) * N_TOK + (q % N_NC)

    mesh = plsc.VectorSubcoreMesh(core_axis_name="c", subcore_axis_name="s")
    run = pl.kernel(
        _tokenizer_body,
        out_type=jax.ShapeDtypeStruct((B * N_TOK, D_TOKEN), jnp.float32),
        mesh=mesh,
        compiler_params=pltpu.CompilerParams(use_tc_tiling_on_sc=False),
        scratch_types=[
            pltpu.VMEM((CH * N_CAT,), jnp.int32),
            pltpu.VMEM((CH * N_CAT,), jnp.int32),
            pltpu.VMEM((CH * N_CAT,), jnp.int32),
            pltpu.VMEM((CH * N_CAT,), jnp.int32),
            pltpu.VMEM((CH * N_NC,), jnp.int32),
            pltpu.VMEM((CH * N_CAT,), jnp.int32),
            pltpu.VMEM((CH * N_NC,), jnp.int32),
            pltpu.VMEM((CH * NUM_PAD,), jnp.float32),
            pltpu.VMEM((NUM_NUMERICAL, D_TOKEN), jnp.float32),
            pltpu.VMEM((NUM_NUMERICAL, D_TOKEN), jnp.float32),
            pltpu.VMEM((D_TOKEN,), jnp.float32),
            pltpu.VMEM((CH * N_CAT, D_TOKEN), jnp.float32),
            pltpu.VMEM((CH * N_NC, D_TOKEN), jnp.float32),
            pltpu.SemaphoreType.DMA,
            pltpu.SemaphoreType.DMA,
        ],
    )
    out_flat = run(x_nump, x_cat_flat, cls_flat, num_weights, num_biases,
                   tables_packed, jnp.asarray(crow_np), jnp.asarray(cpat_np),
                   jnp.asarray(npat_np))
    return out_flat.reshape(B, N_TOK, D_TOKEN)


# speedup vs baseline: 10.1442x; 1.4766x over previous
import functools

import jax
import jax.numpy as jnp
import numpy as np
from jax import lax
from jax.experimental import pallas as pl
from jax.experimental.pallas import tpu as pltpu
from jax.experimental.pallas import tpu_sc as plsc

B = 16384
NUM_NUMERICAL = 13
N_CAT = 26
VOCAB = 100000
D_TOKEN = 32
N_TOK = 1 + NUM_NUMERICAL + N_CAT
N_NC = 1 + NUM_NUMERICAL

NUM_PAD = 16
CH = 64
LANES = 16

RV = 3584
FSTRIDE = 25088
RNB = FSTRIDE // RV
PACKED_ROWS = N_CAT * FSTRIDE * 4


def _repack_body(t0, t1, t2, t3, o_ref):
    x = jnp.concatenate([t0[0], t1[0], t2[0], t3[0]], axis=0)
    o_ref[...] = jnp.transpose(x, (1, 0))


def _repack_table(tbl_t):
    specs = [
        pl.BlockSpec((1, D_TOKEN, RV),
                     lambda c, v, p=p: (c, 0, p * RNB + v))
        for p in range(4)
    ]
    return pl.pallas_call(
        _repack_body,
        grid=(N_CAT, RNB),
        in_specs=specs,
        out_specs=pl.BlockSpec((RV, 128), lambda c, v: (c * RNB + v, 0)),
        out_shape=jax.ShapeDtypeStruct((PACKED_ROWS // 4, 128), jnp.float32),
    )(tbl_t, tbl_t, tbl_t, tbl_t)


def _tokenizer_body(x_num_hbm, x_cat_hbm, cls_hbm, w_hbm, bias_hbm,
                    tables_hbm, crow_hbm, cpat_hbm, npat_hbm, out_hbm,
                    xcat_v, row_v, crow_v, cpat_v, npat_v,
                    cdidx_v, ndidx_v, xnum_v, w_v, b_v, cls_v,
                    cat_v, numcls_v, gsem, ssem):
    info = plsc.get_sparse_core_info()
    nc, ns = info.num_cores, info.num_subcores
    nw = nc * ns
    rows_per_w = B // nw
    nch = rows_per_w // CH
    ids_per_ch = CH * N_CAT
    nc_per_ch = CH * N_NC

    wid = lax.axis_index("s") * nc + lax.axis_index("c")

    pltpu.sync_copy(w_hbm, w_v)
    pltpu.sync_copy(bias_hbm, b_v)
    pltpu.sync_copy(cls_hbm, cls_v)
    pltpu.sync_copy(crow_hbm, crow_v)
    pltpu.sync_copy(cpat_hbm, cpat_v)
    pltpu.sync_copy(npat_hbm, npat_v)

    def chunk_body(k, carry):
        base = (wid * nch + k) * CH

        pltpu.sync_copy(x_cat_hbm.at[pl.ds(base * N_CAT, ids_per_ch)], xcat_v)
        pltpu.sync_copy(x_num_hbm.at[pl.ds(base * NUM_PAD, CH * NUM_PAD)],
                        xnum_v)

        obase = base * N_TOK

        def idx_body(i, _):
            s = pl.ds(i * LANES, LANES)
            v = xcat_v[s]
            one = jnp.full((LANES,), 1, jnp.int32)
            zero = jnp.full((LANES,), 0, jnp.int32)
            p = (jnp.where(v >= FSTRIDE, one, zero)
                 + jnp.where(v >= 2 * FSTRIDE, one, zero)
                 + jnp.where(v >= 3 * FSTRIDE, one, zero))
            row_v[s] = crow_v[s] + ((v - p * FSTRIDE) << 2) + p
            cdidx_v[s] = cpat_v[s] + obase
            return 0

        def nd_body(i, _):
            s = pl.ds(i * LANES, LANES)
            ndidx_v[s] = npat_v[s] + obase
            return 0

        lax.fori_loop(0, ids_per_ch // LANES, idx_body, 0)
        lax.fori_loop(0, nc_per_ch // LANES, nd_body, 0)

        pltpu.async_copy(tables_hbm.at[row_v], cat_v, gsem)

        cls0 = cls_v[pl.ds(0, LANES)]
        cls1 = cls_v[pl.ds(LANES, LANES)]

        def num_body(bi, _):
            r = bi * N_NC
            numcls_v[r, pl.ds(0, LANES)] = cls0
            numcls_v[r, pl.ds(LANES, LANES)] = cls1
            xrow = xnum_v[pl.ds(bi * NUM_PAD, NUM_PAD)]
            for j in range(NUM_NUMERICAL):
                xs = xrow[j]
                numcls_v[r + 1 + j, pl.ds(0, LANES)] = (
                    xs * w_v[j, pl.ds(0, LANES)] + b_v[j, pl.ds(0, LANES)])
                numcls_v[r + 1 + j, pl.ds(LANES, LANES)] = (
                    xs * w_v[j, pl.ds(LANES, LANES)] + b_v[j, pl.ds(LANES, LANES)])
            return 0

        lax.fori_loop(0, CH, num_body, 0)

        pltpu.async_copy(numcls_v, out_hbm.at[ndidx_v], ssem)

        pltpu.make_async_copy(tables_hbm.at[row_v], cat_v, gsem).wait()
        pltpu.async_copy(cat_v, out_hbm.at[cdidx_v], ssem)

        pltpu.make_async_copy(numcls_v, out_hbm.at[ndidx_v], ssem).wait()
        pltpu.make_async_copy(cat_v, out_hbm.at[cdidx_v], ssem).wait()
        return 0

    lax.fori_loop(0, nch, chunk_body, 0)


@functools.partial(jax.jit, static_argnames=())
def kernel(x_num, x_cat, cls_token, num_weights, num_biases, cat_tables):
    tbl_t = jnp.transpose(cat_tables, (0, 2, 1))
    tables_packed = _repack_table(tbl_t).reshape(PACKED_ROWS, D_TOKEN)

    x_nump = jnp.concatenate(
        [x_num, jnp.zeros((B, NUM_PAD - NUM_NUMERICAL), jnp.float32)], axis=1
    ).reshape(B * NUM_PAD)
    x_cat_flat = x_cat.astype(jnp.int32).reshape(B * N_CAT)
    cls_flat = cls_token.reshape(D_TOKEN)

    p = np.arange(CH * N_CAT, dtype=np.int32)
    crow_np = (p % N_CAT) * (4 * FSTRIDE)
    cpat_np = (p // N_CAT) * N_TOK + N_NC + (p % N_CAT)
    q = np.arange(CH * N_NC, dtype=np.int32)
    npat_np = (q // N_NC) * N_TOK + (q % N_NC)

    mesh = plsc.VectorSubcoreMesh(core_axis_name="c", subcore_axis_name="s")
    run = pl.kernel(
        _tokenizer_body,
        out_type=jax.ShapeDtypeStruct((B * N_TOK, D_TOKEN), jnp.float32),
        mesh=mesh,
        compiler_params=pltpu.CompilerParams(use_tc_tiling_on_sc=False),
        scratch_types=[
            pltpu.VMEM((CH * N_CAT,), jnp.int32),
            pltpu.VMEM((CH * N_CAT,), jnp.int32),
            pltpu.VMEM((CH * N_CAT,), jnp.int32),
            pltpu.VMEM((CH * N_CAT,), jnp.int32),
            pltpu.VMEM((CH * N_NC,), jnp.int32),
            pltpu.VMEM((CH * N_CAT,), jnp.int32),
            pltpu.VMEM((CH * N_NC,), jnp.int32),
            pltpu.VMEM((CH * NUM_PAD,), jnp.float32),
            pltpu.VMEM((NUM_NUMERICAL, D_TOKEN), jnp.float32),
            pltpu.VMEM((NUM_NUMERICAL, D_TOKEN), jnp.float32),
            pltpu.VMEM((D_TOKEN,), jnp.float32),
            pltpu.VMEM((CH * N_CAT, D_TOKEN), jnp.float32),
            pltpu.VMEM((CH * N_NC, D_TOKEN), jnp.float32),
            pltpu.SemaphoreType.DMA,
            pltpu.SemaphoreType.DMA,
        ],
    )
    out_flat = run(x_nump, x_cat_flat, cls_flat, num_weights, num_biases,
                   tables_packed, jnp.asarray(crow_np), jnp.asarray(cpat_np),
                   jnp.asarray(npat_np))
    return out_flat.reshape(B, N_TOK, D_TOKEN)


# speedup vs baseline: 13.4258x; 1.3235x over previous
import functools

import jax
import jax.numpy as jnp
import numpy as np
from jax import lax
from jax.experimental import pallas as pl
from jax.experimental.pallas import tpu as pltpu
from jax.experimental.pallas import tpu_sc as plsc

B = 16384
NUM_NUMERICAL = 13
N_CAT = 26
VOCAB = 100000
D_TOKEN = 32
N_TOK = 1 + NUM_NUMERICAL + N_CAT
N_NC = 1 + NUM_NUMERICAL

NUM_PAD = 16
CH = 64
LANES = 16

RV = 3584
FSTRIDE = 25088
RNB = FSTRIDE // RV
PACKED_ROWS = N_CAT * FSTRIDE * 4


def _repack_body(t0, t1, t2, t3, o_ref):
    x = jnp.concatenate([t0[0], t1[0], t2[0], t3[0]], axis=0)
    o_ref[...] = jnp.transpose(x, (1, 0))


def _repack_table(tbl_t):
    specs = [
        pl.BlockSpec((1, D_TOKEN, RV),
                     lambda c, v, p=p: (c, 0, p * RNB + v))
        for p in range(4)
    ]
    return pl.pallas_call(
        _repack_body,
        grid=(N_CAT, RNB),
        in_specs=specs,
        out_specs=pl.BlockSpec((RV, 128), lambda c, v: (c * RNB + v, 0)),
        out_shape=jax.ShapeDtypeStruct((PACKED_ROWS // 4, 128), jnp.float32),
    )(tbl_t, tbl_t, tbl_t, tbl_t)


def _outxpose_body(x_ref, o_ref):
    o_ref[...] = jnp.transpose(x_ref[...], (1, 0))


def _outxpose(flat):
    return pl.pallas_call(
        _outxpose_body,
        grid=(10,),
        in_specs=[pl.BlockSpec((B, 128), lambda g: (0, g))],
        out_specs=pl.BlockSpec((128, B), lambda g: (g, 0)),
        out_shape=jax.ShapeDtypeStruct((N_TOK * D_TOKEN, B), jnp.float32),
        compiler_params=pltpu.CompilerParams(
            vmem_limit_bytes=60 * 1024 * 1024),
    )(flat)


def _tokenizer_body(x_num_hbm, x_cat_hbm, cls_hbm, w_hbm, bias_hbm,
                    tables_hbm, crow_hbm, cpat_hbm, npat_hbm, out_hbm,
                    xcat_v, row_v, crow_v, cpat_v, npat_v,
                    cdidx_v, ndidx_v, xnum_v, w_v, b_v, cls_v,
                    cat_v, numcls_v, gsem, ssem):
    info = plsc.get_sparse_core_info()
    nc, ns = info.num_cores, info.num_subcores
    nw = nc * ns
    rows_per_w = B // nw
    nch = rows_per_w // CH
    ids_per_ch = CH * N_CAT
    nc_per_ch = CH * N_NC

    wid = lax.axis_index("s") * nc + lax.axis_index("c")

    pltpu.sync_copy(w_hbm, w_v)
    pltpu.sync_copy(bias_hbm, b_v)
    pltpu.sync_copy(cls_hbm, cls_v)
    pltpu.sync_copy(crow_hbm, crow_v)
    pltpu.sync_copy(cpat_hbm, cpat_v)
    pltpu.sync_copy(npat_hbm, npat_v)

    def chunk_body(k, carry):
        base = (wid * nch + k) * CH

        pltpu.sync_copy(x_cat_hbm.at[pl.ds(base * N_CAT, ids_per_ch)], xcat_v)
        pltpu.sync_copy(x_num_hbm.at[pl.ds(base * NUM_PAD, CH * NUM_PAD)],
                        xnum_v)

        obase = base * N_TOK

        def idx_body(i, _):
            s = pl.ds(i * LANES, LANES)
            v = xcat_v[s]
            one = jnp.full((LANES,), 1, jnp.int32)
            zero = jnp.full((LANES,), 0, jnp.int32)
            p = (jnp.where(v >= FSTRIDE, one, zero)
                 + jnp.where(v >= 2 * FSTRIDE, one, zero)
                 + jnp.where(v >= 3 * FSTRIDE, one, zero))
            row_v[s] = crow_v[s] + ((v - p * FSTRIDE) << 2) + p
            cdidx_v[s] = cpat_v[s] + obase
            return 0

        def nd_body(i, _):
            s = pl.ds(i * LANES, LANES)
            ndidx_v[s] = npat_v[s] + obase
            return 0

        lax.fori_loop(0, ids_per_ch // LANES, idx_body, 0)
        lax.fori_loop(0, nc_per_ch // LANES, nd_body, 0)

        pltpu.async_copy(tables_hbm.at[row_v], cat_v, gsem)

        cls0 = cls_v[pl.ds(0, LANES)]
        cls1 = cls_v[pl.ds(LANES, LANES)]

        def num_body(bi, _):
            r = bi * N_NC
            numcls_v[r, pl.ds(0, LANES)] = cls0
            numcls_v[r, pl.ds(LANES, LANES)] = cls1
            xrow = xnum_v[pl.ds(bi * NUM_PAD, NUM_PAD)]
            for j in range(NUM_NUMERICAL):
                xs = xrow[j]
                numcls_v[r + 1 + j, pl.ds(0, LANES)] = (
                    xs * w_v[j, pl.ds(0, LANES)] + b_v[j, pl.ds(0, LANES)])
                numcls_v[r + 1 + j, pl.ds(LANES, LANES)] = (
                    xs * w_v[j, pl.ds(LANES, LANES)] + b_v[j, pl.ds(LANES, LANES)])
            return 0

        lax.fori_loop(0, CH, num_body, 0)

        pltpu.async_copy(numcls_v, out_hbm.at[ndidx_v], ssem)

        pltpu.make_async_copy(tables_hbm.at[row_v], cat_v, gsem).wait()
        pltpu.async_copy(cat_v, out_hbm.at[cdidx_v], ssem)

        pltpu.make_async_copy(numcls_v, out_hbm.at[ndidx_v], ssem).wait()
        pltpu.make_async_copy(cat_v, out_hbm.at[cdidx_v], ssem).wait()
        return 0

    lax.fori_loop(0, nch, chunk_body, 0)


@functools.partial(jax.jit, static_argnames=())
def kernel(x_num, x_cat, cls_token, num_weights, num_biases, cat_tables):
    tbl_t = jnp.transpose(cat_tables, (0, 2, 1))
    tables_packed = _repack_table(tbl_t).reshape(PACKED_ROWS, D_TOKEN)

    x_nump = jnp.concatenate(
        [x_num, jnp.zeros((B, NUM_PAD - NUM_NUMERICAL), jnp.float32)], axis=1
    ).reshape(B * NUM_PAD)
    x_cat_flat = x_cat.astype(jnp.int32).reshape(B * N_CAT)
    cls_flat = cls_token.reshape(D_TOKEN)

    p = np.arange(CH * N_CAT, dtype=np.int32)
    crow_np = (p % N_CAT) * (4 * FSTRIDE)
    cpat_np = (p // N_CAT) * N_TOK + N_NC + (p % N_CAT)
    q = np.arange(CH * N_NC, dtype=np.int32)
    npat_np = (q // N_NC) * N_TOK + (q % N_NC)

    mesh = plsc.VectorSubcoreMesh(core_axis_name="c", subcore_axis_name="s")
    run = pl.kernel(
        _tokenizer_body,
        out_type=jax.ShapeDtypeStruct((B * N_TOK, D_TOKEN), jnp.float32),
        mesh=mesh,
        compiler_params=pltpu.CompilerParams(use_tc_tiling_on_sc=False),
        scratch_types=[
            pltpu.VMEM((CH * N_CAT,), jnp.int32),
            pltpu.VMEM((CH * N_CAT,), jnp.int32),
            pltpu.VMEM((CH * N_CAT,), jnp.int32),
            pltpu.VMEM((CH * N_CAT,), jnp.int32),
            pltpu.VMEM((CH * N_NC,), jnp.int32),
            pltpu.VMEM((CH * N_CAT,), jnp.int32),
            pltpu.VMEM((CH * N_NC,), jnp.int32),
            pltpu.VMEM((CH * NUM_PAD,), jnp.float32),
            pltpu.VMEM((NUM_NUMERICAL, D_TOKEN), jnp.float32),
            pltpu.VMEM((NUM_NUMERICAL, D_TOKEN), jnp.float32),
            pltpu.VMEM((D_TOKEN,), jnp.float32),
            pltpu.VMEM((CH * N_CAT, D_TOKEN), jnp.float32),
            pltpu.VMEM((CH * N_NC, D_TOKEN), jnp.float32),
            pltpu.SemaphoreType.DMA,
            pltpu.SemaphoreType.DMA,
        ],
    )
    out_flat = run(x_nump, x_cat_flat, cls_flat, num_weights, num_biases,
                   tables_packed, jnp.asarray(crow_np), jnp.asarray(cpat_np),
                   jnp.asarray(npat_np))
    out_t = _outxpose(out_flat.reshape(B, N_TOK * D_TOKEN))
    return jnp.transpose(
        out_t.reshape(N_TOK, D_TOKEN, B), (2, 0, 1))
